# Initial kernel scaffold; baseline (speedup 1.0000x reference)
import jax
import jax.numpy as jnp
from jax.experimental import pallas as pl


def kernel(total_token_num, expert_start_loc, recv_x, recv_x_scale, recv_topk, output_tensor, output_tensor_scale, output_index):
    raise NotImplementedError("write your pallas kernel here")


# speedup vs baseline: 2.3906x; 2.3906x over previous
import functools

import jax
import jax.numpy as jnp
from jax import lax
from jax.experimental import pallas as pl
from jax.experimental.pallas import tpu as pltpu
from jax.experimental.pallas import tpu_sc as plsc

_T = 16384
_H = 2048
_SH = 16
_K = 2
_E = 16
_N = _T * _K
_NC = 2
_NS = 16
_NW = _NC * _NS
_CHUNK = _N // _NW
_TOK = _T // _NW
_BT = 16
_NB = _TOK // _BT
_VREGS = _CHUNK // 16


def _body(topk_hbm, starts_hbm, x_hbm, xs_hbm,
          out_hbm, outs_hbm, oidx_hbm,
          hist_ids_v, own_ids_v, dest_v, cnt_v, hist_v, hist_all_v, starts_v,
          buf, buf_s, hist_sh, sem):
    c = lax.axis_index("c")
    s = lax.axis_index("s")
    wid = c * _NS + s
    lanes = lax.iota(jnp.int32, 16)

    pltpu.sync_copy(topk_hbm.at[pl.ds(s * 2 * _CHUNK, 2 * _CHUNK)], hist_ids_v)
    for j in range(2):
        def hist_step(r, acc, j=j):
            v = hist_ids_v[pl.ds(j * _CHUNK + r * 16, 16)]
            for e in range(_E):
                tot = jnp.sum((v == e).astype(jnp.int32))
                acc = jnp.where(lanes == e, acc + tot, acc)
            return acc
        hist_v[...] = lax.fori_loop(0, _VREGS, hist_step,
                                    jnp.zeros((16,), jnp.int32))
        pltpu.sync_copy(hist_v, hist_sh.at[2 * s + j])

    plsc.subcore_barrier()

    pltpu.sync_copy(hist_sh, hist_all_v)
    pltpu.sync_copy(starts_hbm, starts_v)
    base = lax.fori_loop(0, wid, lambda w, acc: acc + hist_all_v[w],
                         starts_v[...])
    cnt_v[...] = base

    pltpu.sync_copy(topk_hbm.at[pl.ds(wid * _CHUNK, _CHUNK)], own_ids_v)

    def rank_step(r, _):
        v = own_ids_v[pl.ds(r * 16, 16)]
        g = plsc.load_gather(cnt_v, [v])
        rank = jnp.zeros((16,), jnp.int32)
        vc = jnp.zeros((16,), jnp.int32)
        for e in range(_E):
            m = v == e
            cum = jnp.cumsum(m.astype(jnp.int32))
            rank = jnp.where(m, cum - 1, rank)
            tot = jnp.sum(m.astype(jnp.int32))
            vc = jnp.where(lanes == e, vc + tot, vc)
        dest_v[pl.ds(r * 16, 16)] = g + rank
        cnt_v[...] = cnt_v[...] + vc
        return 0

    lax.fori_loop(0, _VREGS, rank_step, 0)

    pltpu.sync_copy(dest_v, oidx_hbm.at[pl.ds(wid * _CHUNK, _CHUNK)])

    def copy_step(b, _):
        tok0 = wid * _TOK + b * _BT
        pltpu.sync_copy(x_hbm.at[pl.ds(tok0, _BT)], buf)
        pltpu.sync_copy(xs_hbm.at[pl.ds(tok0, _BT)], buf_s)
        idx_a = plsc.load_gather(dest_v, [b * (2 * _BT) + 2 * lanes])
        idx_b = plsc.load_gather(dest_v, [b * (2 * _BT) + 2 * lanes + 1])
        h1 = pltpu.async_copy(buf, out_hbm.at[idx_a], sem)
        h2 = pltpu.async_copy(buf, out_hbm.at[idx_b], sem)
        h3 = pltpu.async_copy(buf_s, outs_hbm.at[idx_a], sem)
        h4 = pltpu.async_copy(buf_s, outs_hbm.at[idx_b], sem)
        h1.wait(); h2.wait(); h3.wait(); h4.wait()
        return 0

    lax.fori_loop(0, _NB, copy_step, 0)


_dispatch = pl.kernel(
    _body,
    out_type=[
        jax.ShapeDtypeStruct((_N, _H), jnp.float32),
        jax.ShapeDtypeStruct((_N, _SH), jnp.float32),
        jax.ShapeDtypeStruct((_N,), jnp.int32),
    ],
    mesh=plsc.VectorSubcoreMesh(core_axis_name="c", subcore_axis_name="s",
                                num_cores=_NC, num_subcores=_NS),
    scratch_types=[
        pltpu.VMEM((2 * _CHUNK,), jnp.int32),
        pltpu.VMEM((_CHUNK,), jnp.int32),
        pltpu.VMEM((_CHUNK,), jnp.int32),
        pltpu.VMEM((16,), jnp.int32),
        pltpu.VMEM((16,), jnp.int32),
        pltpu.VMEM((_NW, 16), jnp.int32),
        pltpu.VMEM((16,), jnp.int32),
        pltpu.VMEM((_BT, _H), jnp.float32),
        pltpu.VMEM((_BT, _SH), jnp.float32),
        pltpu.VMEM_SHARED((_NW, 16), jnp.int32),
        pltpu.SemaphoreType.DMA,
    ],
    compiler_params=pltpu.CompilerParams(needs_layout_passes=False,
                                         use_tc_tiling_on_sc=False),
)


def kernel(total_token_num, expert_start_loc, recv_x, recv_x_scale, recv_topk,
           output_tensor, output_tensor_scale, output_index):
    del total_token_num, output_tensor, output_tensor_scale
    topk_flat = recv_topk.reshape(-1)
    out, out_s, oidx_flat = _dispatch(
        topk_flat, expert_start_loc.astype(jnp.int32), recv_x, recv_x_scale)
    return out, out_s, oidx_flat.reshape(output_index.shape)


# speedup vs baseline: 2.5058x; 1.0482x over previous
import functools

import jax
import jax.numpy as jnp
from jax import lax
from jax.experimental import pallas as pl
from jax.experimental.pallas import tpu as pltpu
from jax.experimental.pallas import tpu_sc as plsc

_T = 16384
_H = 2048
_SH = 16
_K = 2
_E = 16
_N = _T * _K
_NC = 2
_NS = 16
_NW = _NC * _NS
_CHUNK = _N // _NW
_TOK = _T // _NW
_BT = 16
_NB = _TOK // _BT
_VREGS = _CHUNK // 16


def _body(topk_hbm, starts_hbm, x_hbm, xs_hbm,
          out_hbm, outs_hbm, oidx_hbm,
          hist_ids_v, own_ids_v, dest_v, cnt_v, hist_v, hist_all_v, starts_v,
          buf, buf2, buf_s, buf_s2, hist_sh, sem, sem_in):
    c = lax.axis_index("c")
    s = lax.axis_index("s")
    wid = c * _NS + s
    lanes = lax.iota(jnp.int32, 16)

    pltpu.sync_copy(topk_hbm.at[pl.ds(s * 2 * _CHUNK, 2 * _CHUNK)], hist_ids_v)
    for j in range(2):
        def hist_step(r, acc, j=j):
            v = hist_ids_v[pl.ds(j * _CHUNK + r * 16, 16)]
            for e in range(_E):
                tot = jnp.sum((v == e).astype(jnp.int32))
                acc = jnp.where(lanes == e, acc + tot, acc)
            return acc
        hist_v[...] = lax.fori_loop(0, _VREGS, hist_step,
                                    jnp.zeros((16,), jnp.int32))
        pltpu.sync_copy(hist_v, hist_sh.at[2 * s + j])

    plsc.subcore_barrier()

    pltpu.sync_copy(hist_sh, hist_all_v)
    pltpu.sync_copy(starts_hbm, starts_v)
    base = lax.fori_loop(0, wid, lambda w, acc: acc + hist_all_v[w],
                         starts_v[...])
    cnt_v[...] = base

    pltpu.sync_copy(topk_hbm.at[pl.ds(wid * _CHUNK, _CHUNK)], own_ids_v)

    def rank_step(r, _):
        v = own_ids_v[pl.ds(r * 16, 16)]
        g = plsc.load_gather(cnt_v, [v])
        rank = jnp.zeros((16,), jnp.int32)
        vc = jnp.zeros((16,), jnp.int32)
        for e in range(_E):
            m = v == e
            cum = jnp.cumsum(m.astype(jnp.int32))
            rank = jnp.where(m, cum - 1, rank)
            tot = jnp.sum(m.astype(jnp.int32))
            vc = jnp.where(lanes == e, vc + tot, vc)
        dest_v[pl.ds(r * 16, 16)] = g + rank
        cnt_v[...] = cnt_v[...] + vc
        return 0

    lax.fori_loop(0, _VREGS, rank_step, 0)

    pltpu.sync_copy(dest_v, oidx_hbm.at[pl.ds(wid * _CHUNK, _CHUNK)])

    bufs = (buf, buf2)
    sbufs = (buf_s, buf_s2)
    h_in = [None, None]
    h_sin = [None, None]
    h_out = [None, None]

    def start_in(b):
        i = b % 2
        tok0 = wid * _TOK + b * _BT
        h_in[i] = pltpu.async_copy(x_hbm.at[pl.ds(tok0, _BT)], bufs[i], sem_in)
        h_sin[i] = pltpu.async_copy(xs_hbm.at[pl.ds(tok0, _BT)], sbufs[i],
                                    sem_in)

    start_in(0)
    for b in range(_NB):
        i = b % 2
        if b + 1 < _NB:
            if h_out[1 - i] is not None:
                for h in h_out[1 - i]:
                    h.wait()
            start_in(b + 1)
        h_in[i].wait()
        h_sin[i].wait()
        idx_a = plsc.load_gather(dest_v, [b * (2 * _BT) + 2 * lanes])
        idx_b = plsc.load_gather(dest_v, [b * (2 * _BT) + 2 * lanes + 1])
        h_out[i] = (
            pltpu.async_copy(bufs[i], out_hbm.at[idx_a], sem),
            pltpu.async_copy(bufs[i], out_hbm.at[idx_b], sem),
            pltpu.async_copy(sbufs[i], outs_hbm.at[idx_a], sem),
            pltpu.async_copy(sbufs[i], outs_hbm.at[idx_b], sem),
        )
    for hs in h_out:
        if hs is not None:
            for h in hs:
                h.wait()


_dispatch = pl.kernel(
    _body,
    out_type=[
        jax.ShapeDtypeStruct((_N, _H), jnp.float32),
        jax.ShapeDtypeStruct((_N, _SH), jnp.float32),
        jax.ShapeDtypeStruct((_N,), jnp.int32),
    ],
    mesh=plsc.VectorSubcoreMesh(core_axis_name="c", subcore_axis_name="s",
                                num_cores=_NC, num_subcores=_NS),
    scratch_types=[
        pltpu.VMEM((2 * _CHUNK,), jnp.int32),
        pltpu.VMEM((_CHUNK,), jnp.int32),
        pltpu.VMEM((_CHUNK,), jnp.int32),
        pltpu.VMEM((16,), jnp.int32),
        pltpu.VMEM((16,), jnp.int32),
        pltpu.VMEM((_NW, 16), jnp.int32),
        pltpu.VMEM((16,), jnp.int32),
        pltpu.VMEM((_BT, _H), jnp.float32),
        pltpu.VMEM((_BT, _H), jnp.float32),
        pltpu.VMEM((_BT, _SH), jnp.float32),
        pltpu.VMEM((_BT, _SH), jnp.float32),
        pltpu.VMEM_SHARED((_NW, 16), jnp.int32),
        pltpu.SemaphoreType.DMA,
        pltpu.SemaphoreType.DMA,
    ],
    compiler_params=pltpu.CompilerParams(needs_layout_passes=False,
                                         use_tc_tiling_on_sc=False),
)


def kernel(total_token_num, expert_start_loc, recv_x, recv_x_scale, recv_topk,
           output_tensor, output_tensor_scale, output_index):
    del total_token_num, output_tensor, output_tensor_scale
    topk_flat = recv_topk.reshape(-1)
    out, out_s, oidx_flat = _dispatch(
        topk_flat, expert_start_loc.astype(jnp.int32), recv_x, recv_x_scale)
    return out, out_s, oidx_flat.reshape(output_index.shape)


# speedup vs baseline: 6.2343x; 2.4880x over previous
import jax
import jax.numpy as jnp
from jax import lax
from jax.experimental import pallas as pl
from jax.experimental.pallas import tpu as pltpu
from jax.experimental.pallas import tpu_sc as plsc

_T = 16384
_H = 2048
_SH = 16
_K = 2
_E = 16
_N = _T * _K
_NC = 2
_NS = 16
_NW = _NC * _NS
_CHUNK = _N // _NW
_TOK = _T // _NW
_BT = 16
_NB = _TOK // _BT
_VREGS = _CHUNK // 16


def _main_body(topk_hbm, starts_hbm, x_hbm,
               out_hbm, oidx_hbm,
               hist_ids_v, own_ids_v, dest_v, cnt_v, hist_v, hist_all_v,
               starts_v, buf, buf2, hist_sh, sem, sem_in):
    c = lax.axis_index("c")
    s = lax.axis_index("s")
    wid = c * _NS + s
    lanes = lax.iota(jnp.int32, 16)

    pltpu.sync_copy(topk_hbm.at[pl.ds(s * 2 * _CHUNK, 2 * _CHUNK)], hist_ids_v)
    for j in range(2):
        def hist_step(r, acc, j=j):
            v = hist_ids_v[pl.ds(j * _CHUNK + r * 16, 16)]
            for e in range(_E):
                tot = jnp.sum((v == e).astype(jnp.int32))
                acc = jnp.where(lanes == e, acc + tot, acc)
            return acc
        hist_v[...] = lax.fori_loop(0, _VREGS, hist_step,
                                    jnp.zeros((16,), jnp.int32))
        pltpu.sync_copy(hist_v, hist_sh.at[2 * s + j])

    plsc.subcore_barrier()

    pltpu.sync_copy(hist_sh, hist_all_v)
    pltpu.sync_copy(starts_hbm, starts_v)
    base = lax.fori_loop(0, wid, lambda w, acc: acc + hist_all_v[w],
                         starts_v[...])
    cnt_v[...] = base

    pltpu.sync_copy(topk_hbm.at[pl.ds(wid * _CHUNK, _CHUNK)], own_ids_v)

    def rank_step(r, _):
        v = own_ids_v[pl.ds(r * 16, 16)]
        g = plsc.load_gather(cnt_v, [v])
        rank = jnp.zeros((16,), jnp.int32)
        vc = jnp.zeros((16,), jnp.int32)
        for e in range(_E):
            m = v == e
            cum = jnp.cumsum(m.astype(jnp.int32))
            rank = jnp.where(m, cum - 1, rank)
            tot = jnp.sum(m.astype(jnp.int32))
            vc = jnp.where(lanes == e, vc + tot, vc)
        dest_v[pl.ds(r * 16, 16)] = g + rank
        cnt_v[...] = cnt_v[...] + vc
        return 0

    lax.fori_loop(0, _VREGS, rank_step, 0)

    pltpu.sync_copy(dest_v, oidx_hbm.at[pl.ds(wid * _CHUNK, _CHUNK)])

    bufs = (buf, buf2)
    h_in = [None, None]
    h_out = [None, None]

    def start_in(b):
        i = b % 2
        tok0 = wid * _TOK + b * _BT
        h_in[i] = pltpu.async_copy(x_hbm.at[pl.ds(tok0, _BT)], bufs[i], sem_in)

    start_in(0)
    for b in range(_NB):
        i = b % 2
        if b + 1 < _NB:
            if h_out[1 - i] is not None:
                for h in h_out[1 - i]:
                    h.wait()
            start_in(b + 1)
        h_in[i].wait()
        idx_a = plsc.load_gather(dest_v, [b * (2 * _BT) + 2 * lanes])
        idx_b = plsc.load_gather(dest_v, [b * (2 * _BT) + 2 * lanes + 1])
        h_out[i] = (
            pltpu.async_copy(bufs[i], out_hbm.at[idx_a], sem),
            pltpu.async_copy(bufs[i], out_hbm.at[idx_b], sem),
        )
    for hs in h_out:
        if hs is not None:
            for h in hs:
                h.wait()


_dispatch_main = pl.kernel(
    _main_body,
    out_type=[
        jax.ShapeDtypeStruct((_N, _H), jnp.float32),
        jax.ShapeDtypeStruct((_N,), jnp.int32),
    ],
    mesh=plsc.VectorSubcoreMesh(core_axis_name="c", subcore_axis_name="s",
                                num_cores=_NC, num_subcores=_NS),
    scratch_types=[
        pltpu.VMEM((2 * _CHUNK,), jnp.int32),
        pltpu.VMEM((_CHUNK,), jnp.int32),
        pltpu.VMEM((_CHUNK,), jnp.int32),
        pltpu.VMEM((16,), jnp.int32),
        pltpu.VMEM((16,), jnp.int32),
        pltpu.VMEM((_NW, 16), jnp.int32),
        pltpu.VMEM((16,), jnp.int32),
        pltpu.VMEM((_BT, _H), jnp.float32),
        pltpu.VMEM((_BT, _H), jnp.float32),
        pltpu.VMEM_SHARED((_NW, 16), jnp.int32),
        pltpu.SemaphoreType.DMA,
        pltpu.SemaphoreType.DMA,
    ],
    compiler_params=pltpu.CompilerParams(needs_layout_passes=False),
)


def _scale_body(xs_hbm, dests_hbm, outs_hbm,
                dest_v, sbuf, idx_a_v, idx_b_v, sem):
    c = lax.axis_index("c")
    s = lax.axis_index("s")
    wid = c * _NS + s
    lanes = lax.iota(jnp.int32, 16)

    pltpu.sync_copy(dests_hbm.at[pl.ds(wid * _CHUNK, _CHUNK)], dest_v)
    pltpu.sync_copy(xs_hbm.at[pl.ds(wid * _TOK, _TOK)], sbuf)

    for j in range(4):
        for i in range(8):
            base = 256 * j + 32 * i
            idx_a_v[j, pl.ds(16 * i, 16)] = plsc.load_gather(
                dest_v, [base + 2 * lanes])
            idx_b_v[j, pl.ds(16 * i, 16)] = plsc.load_gather(
                dest_v, [base + 2 * lanes + 1])

    hs = []
    for j in range(4):
        src = sbuf.at[pl.ds(128 * j, 128)]
        hs.append(pltpu.async_copy(src, outs_hbm.at[idx_a_v.at[j]], sem))
        hs.append(pltpu.async_copy(src, outs_hbm.at[idx_b_v.at[j]], sem))
    for h in hs:
        h.wait()


_dispatch_scale = pl.kernel(
    _scale_body,
    out_type=[
        jax.ShapeDtypeStruct((_N, _SH), jnp.float32),
    ],
    mesh=plsc.VectorSubcoreMesh(core_axis_name="c", subcore_axis_name="s",
                                num_cores=_NC, num_subcores=_NS),
    scratch_types=[
        pltpu.VMEM((_CHUNK,), jnp.int32),
        pltpu.VMEM((_TOK, _SH), jnp.float32),
        pltpu.VMEM((4, 128), jnp.int32),
        pltpu.VMEM((4, 128), jnp.int32),
        pltpu.SemaphoreType.DMA,
    ],
    compiler_params=pltpu.CompilerParams(needs_layout_passes=False,
                                         use_tc_tiling_on_sc=False),
)


def kernel(total_token_num, expert_start_loc, recv_x, recv_x_scale, recv_topk,
           output_tensor, output_tensor_scale, output_index):
    del total_token_num, output_tensor, output_tensor_scale
    topk_flat = recv_topk.reshape(-1)
    out, oidx_flat = _dispatch_main(
        topk_flat, expert_start_loc.astype(jnp.int32), recv_x)
    (out_s,) = _dispatch_scale(recv_x_scale, oidx_flat)
    return out, out_s, oidx_flat.reshape(output_index.shape)
